# baseline (speedup 1.0000x reference)
import functools

import jax
import jax.numpy as jnp
from jax import lax
from jax.experimental import pallas as pl
from jax.experimental.pallas import tpu as pltpu
from jax.experimental.pallas import tpu_sc as plsc

NC = 2
NS = 16
NW = NC * NS
LANES = 16


@functools.partial(jax.jit, static_argnames=("n",))
def _sc_add(xf, pf, n):
    per_w = n // NW
    chunk = min(per_w, 16384)
    n_chunks = per_w // chunk

    mesh = plsc.VectorSubcoreMesh(core_axis_name="c", subcore_axis_name="s")

    @functools.partial(
        pl.kernel,
        out_type=jax.ShapeDtypeStruct((n,), jnp.float32),
        mesh=mesh,
        scratch_types=[
            pltpu.VMEM((chunk,), jnp.float32),
            pltpu.VMEM((chunk,), jnp.float32),
        ],
    )
    def k(x_hbm, p_hbm, o_hbm, x_v, p_v):
        wid = lax.axis_index("s") * NC + lax.axis_index("c")
        base = wid * per_w
        for c in range(n_chunks):
            off = base + c * chunk
            pltpu.sync_copy(x_hbm.at[pl.ds(off, chunk)], x_v)
            pltpu.sync_copy(p_hbm.at[pl.ds(off, chunk)], p_v)

            def body(i, _):
                s = pl.ds(i * LANES, LANES)
                x_v[s] = x_v[s] + p_v[s]
                return 0

            lax.fori_loop(0, chunk // LANES, body, 0, unroll=8)
            pltpu.sync_copy(x_v, o_hbm.at[pl.ds(off, chunk)])

    return k(xf, pf)


def kernel(x, pe):
    s, d = x.shape
    out = _sc_add(x.reshape(-1), pe[:s].reshape(-1), s * d)
    return out.reshape(s, d)


# speedup vs baseline: 1.0525x; 1.0525x over previous
import functools

import jax
import jax.numpy as jnp
from jax import lax
from jax.experimental import pallas as pl
from jax.experimental.pallas import tpu as pltpu
from jax.experimental.pallas import tpu_sc as plsc

NC = 2
NS = 16
NW = NC * NS
LANES = 16
NBUF = 2


@functools.partial(jax.jit, static_argnames=("n",))
def _sc_add(xf, pf, n):
    per_w = n // NW
    chunk = min(per_w, 16384)
    n_chunks = per_w // chunk

    mesh = plsc.VectorSubcoreMesh(core_axis_name="c", subcore_axis_name="s")

    @functools.partial(
        pl.kernel,
        out_type=jax.ShapeDtypeStruct((n,), jnp.float32),
        mesh=mesh,
        scratch_types=[
            pltpu.VMEM((NBUF, chunk), jnp.float32),
            pltpu.VMEM((NBUF, chunk), jnp.float32),
            pltpu.SemaphoreType.DMA((NBUF,)),
            pltpu.SemaphoreType.DMA((NBUF,)),
        ],
    )
    def k(x_hbm, p_hbm, o_hbm, x_v, p_v, ld_sem, st_sem):
        wid = lax.axis_index("s") * NC + lax.axis_index("c")
        base = wid * per_w

        loads = {}
        stores = {}

        def start_load(c):
            b = c % NBUF
            off = base + c * chunk
            loads[c] = (
                pltpu.make_async_copy(
                    x_hbm.at[pl.ds(off, chunk)], x_v.at[b], ld_sem.at[b]
                ),
                pltpu.make_async_copy(
                    p_hbm.at[pl.ds(off, chunk)], p_v.at[b], ld_sem.at[b]
                ),
            )
            loads[c][0].start()
            loads[c][1].start()

        start_load(0)
        if n_chunks > 1:
            start_load(1)

        for c in range(n_chunks):
            b = c % NBUF
            for d in loads.pop(c):
                d.wait()

            def body(i, _):
                s = pl.ds(i * LANES, LANES)
                x_v[b, s] = x_v[b, s] + p_v[b, s]
                return 0

            lax.fori_loop(0, chunk // LANES, body, 0, unroll=8)

            off = base + c * chunk
            stores[c] = pltpu.make_async_copy(
                x_v.at[b], o_hbm.at[pl.ds(off, chunk)], st_sem.at[b]
            )
            stores[c].start()

            nxt = c + NBUF
            if nxt < n_chunks:
                stores.pop(nxt - NBUF).wait()
                start_load(nxt)

        for d in stores.values():
            d.wait()

    return k(xf, pf)


def kernel(x, pe):
    s, d = x.shape
    out = _sc_add(x.reshape(-1), pe[:s].reshape(-1), s * d)
    return out.reshape(s, d)


# speedup vs baseline: 1.4631x; 1.3901x over previous
import functools

import jax
import jax.numpy as jnp
from jax import lax
from jax.experimental import pallas as pl
from jax.experimental.pallas import tpu as pltpu
from jax.experimental.pallas import tpu_sc as plsc

NC = 2
NS = 16
NW = NC * NS
LANES = 16
NBUF = 2
CHUNK_ROWS = 256


@jax.jit
def _sc_add(x, pe):
    seq, d = x.shape
    rows_per_w = seq // NW
    chunk = min(rows_per_w, CHUNK_ROWS)
    n_chunks = rows_per_w // chunk
    vecs_per_row = d // LANES

    mesh = plsc.VectorSubcoreMesh(core_axis_name="c", subcore_axis_name="s")

    @functools.partial(
        pl.kernel,
        out_type=jax.ShapeDtypeStruct((seq, d), jnp.float32),
        mesh=mesh,
        scratch_types=[
            pltpu.VMEM((NBUF, chunk, d), jnp.float32),
            pltpu.VMEM((NBUF, chunk, d), jnp.float32),
            pltpu.SemaphoreType.DMA((NBUF,)),
            pltpu.SemaphoreType.DMA((NBUF,)),
        ],
    )
    def k(x_hbm, p_hbm, o_hbm, x_v, p_v, ld_sem, st_sem):
        wid = lax.axis_index("s") * NC + lax.axis_index("c")
        base = wid * rows_per_w

        loads = {}
        stores = {}

        def start_load(c):
            b = c % NBUF
            off = base + c * chunk
            loads[c] = (
                pltpu.make_async_copy(
                    x_hbm.at[pl.ds(off, chunk), :], x_v.at[b], ld_sem.at[b]
                ),
                pltpu.make_async_copy(
                    p_hbm.at[pl.ds(off, chunk), :], p_v.at[b], ld_sem.at[b]
                ),
            )
            loads[c][0].start()
            loads[c][1].start()

        start_load(0)
        if n_chunks > 1:
            start_load(1)

        for c in range(n_chunks):
            b = c % NBUF
            for dsc in loads.pop(c):
                dsc.wait()

            def body(r, _):
                for j in range(vecs_per_row):
                    s = pl.ds(j * LANES, LANES)
                    x_v[b, r, s] = x_v[b, r, s] + p_v[b, r, s]
                return 0

            lax.fori_loop(0, chunk, body, 0, unroll=2)

            off = base + c * chunk
            stores[c] = pltpu.make_async_copy(
                x_v.at[b], o_hbm.at[pl.ds(off, chunk), :], st_sem.at[b]
            )
            stores[c].start()

            nxt = c + NBUF
            if nxt < n_chunks:
                stores.pop(nxt - NBUF).wait()
                start_load(nxt)

        for dsc in stores.values():
            dsc.wait()

    return k(x, pe)


def kernel(x, pe):
    return _sc_add(x, pe[: x.shape[0]])
